# Initial kernel scaffold; baseline (speedup 1.0000x reference)
import jax
import jax.numpy as jnp
from jax.experimental import pallas as pl


def kernel(input_ids, orig_table, new_table):
    raise NotImplementedError("write your pallas kernel here")


# speedup vs baseline: 13.1033x; 13.1033x over previous
import functools

import jax
import jax.numpy as jnp
from jax import lax
from jax.experimental import pallas as pl
from jax.experimental.pallas import tpu as pltpu
from jax.experimental.pallas import tpu_sc as plsc

SUB = 128
NSUB = 5
CHUNK = SUB * NSUB


def _make_kernel(n_rows, vocab, n_new, dim):
    info = plsc.get_sparse_core_info()
    nc, ns = info.num_cores, info.num_subcores
    nw = nc * ns
    assert n_rows % (nw * CHUNK) == 0
    per_w = n_rows // nw
    n_chunks = per_w // CHUNK
    assert dim % 16 == 0

    mesh = plsc.VectorSubcoreMesh(core_axis_name="c", subcore_axis_name="s")

    @functools.partial(
        pl.kernel,
        mesh=mesh,
        out_type=jax.ShapeDtypeStruct((n_rows, dim), jnp.float32),
        scratch_types=[
            pltpu.VMEM((CHUNK,), jnp.int32),
            pltpu.VMEM((NSUB, SUB), jnp.int32),
            pltpu.VMEM((CHUNK, dim), jnp.float32),
            pltpu.VMEM((n_new, dim), jnp.float32),
            pltpu.SemaphoreType.DMA,
        ],
    )
    def k(ids_hbm, orig_hbm, new_hbm, out_hbm, idx_v, safe_v, buf, new_v, sem):
        wid = lax.axis_index("s") * nc + lax.axis_index("c")
        pltpu.sync_copy(new_hbm, new_v)

        def chunk_body(t, carry):
            base = wid * per_w + t * CHUNK
            pltpu.sync_copy(ids_hbm.at[pl.ds(base, CHUNK)], idx_v)
            acc = None
            for g in range(CHUNK // 16):
                v = idx_v[pl.ds(g * 16, 16)]
                acc = v if acc is None else jnp.maximum(acc, v)
                safe = jnp.where(v >= vocab, 0, v)
                safe_v[g * 16 // SUB, pl.ds((g * 16) % SUB, 16)] = safe
            handles = [
                pltpu.async_copy(
                    orig_hbm.at[safe_v.at[j]],
                    buf.at[pl.ds(j * SUB, SUB)],
                    sem,
                )
                for j in range(NSUB)
            ]
            for h in handles:
                h.wait()
            pltpu.sync_copy(buf, out_hbm.at[pl.ds(base, CHUNK)])

            for lane in range(16):

                @pl.when(acc[lane] >= vocab)
                def _(lane=lane):
                    def scan_groups(g, cc):
                        v = idx_v[pl.ds(g * 16, 16)]
                        s = v[lane]

                        @pl.when(s >= vocab)
                        def _():
                            nrow = s - vocab
                            pltpu.sync_copy(
                                new_v.at[pl.ds(nrow, 1)],
                                out_hbm.at[pl.ds(base + g * 16 + lane, 1)],
                            )

                        return cc

                    lax.fori_loop(0, CHUNK // 16, scan_groups, 0)

            return carry

        lax.fori_loop(0, n_chunks, chunk_body, 0)

    return k


def kernel(input_ids, orig_table, new_table):
    b, l = input_ids.shape
    vocab, dim = orig_table.shape
    n_new = new_table.shape[0]
    ids = input_ids.reshape(-1).astype(jnp.int32)
    k = _make_kernel(b * l, vocab, n_new, dim)
    out = k(ids, orig_table, new_table)
    return out.reshape(b, l, dim)


# speedup vs baseline: 13.9409x; 1.0639x over previous
import functools

import jax
import jax.numpy as jnp
from jax import lax
from jax.experimental import pallas as pl
from jax.experimental.pallas import tpu as pltpu
from jax.experimental.pallas import tpu_sc as plsc

SUB = 128
NSUB = 2
CHUNK = SUB * NSUB


def _make_kernel(n_rows, vocab, n_new, dim):
    info = plsc.get_sparse_core_info()
    nc, ns = info.num_cores, info.num_subcores
    nw = nc * ns
    assert n_rows % (nw * CHUNK) == 0
    per_w = n_rows // nw
    n_chunks = per_w // CHUNK
    assert n_chunks % 2 == 1 and n_chunks >= 3
    n_pairs = (n_chunks - 1) // 2
    assert dim % 16 == 0

    mesh = plsc.VectorSubcoreMesh(core_axis_name="c", subcore_axis_name="s")

    @functools.partial(
        pl.kernel,
        mesh=mesh,
        out_type=jax.ShapeDtypeStruct((n_rows, dim), jnp.float32),
        scratch_types=[
            pltpu.VMEM((CHUNK,), jnp.int32),
            pltpu.VMEM((CHUNK,), jnp.int32),
            pltpu.VMEM((NSUB, SUB), jnp.int32),
            pltpu.VMEM((NSUB, SUB), jnp.int32),
            pltpu.VMEM((CHUNK, dim), jnp.float32),
            pltpu.VMEM((CHUNK, dim), jnp.float32),
            pltpu.VMEM((n_new, dim), jnp.float32),
            pltpu.SemaphoreType.DMA,
            pltpu.SemaphoreType.DMA,
        ],
    )
    def k(ids_hbm, orig_hbm, new_hbm, out_hbm,
          idx0, idx1, safe0, safe1, buf0, buf1, new_v, sem0, sem1):
        wid = lax.axis_index("s") * nc + lax.axis_index("c")
        w_base = wid * per_w
        pltpu.sync_copy(new_hbm, new_v)

        def stage(t, idx, safe):
            pltpu.sync_copy(ids_hbm.at[pl.ds(w_base + t * CHUNK, CHUNK)], idx)
            acc = None
            for g in range(CHUNK // 16):
                v = idx[pl.ds(g * 16, 16)]
                acc = v if acc is None else jnp.maximum(acc, v)
                safe[g * 16 // SUB, pl.ds((g * 16) % SUB, 16)] = jnp.where(
                    v >= vocab, 0, v
                )
            return acc

        def fire(safe, buf, sem):
            for j in range(NSUB):
                pltpu.async_copy(
                    orig_hbm.at[safe.at[j]], buf.at[pl.ds(j * SUB, SUB)], sem
                )

        def wait_gather(buf, sem):
            pltpu.make_async_copy(orig_hbm.at[pl.ds(0, CHUNK)], buf, sem).wait()

        def patch(t, idx, acc):
            base = w_base + t * CHUNK
            for lane in range(16):

                @pl.when(acc[lane] >= vocab)
                def _(lane=lane):
                    def scan_groups(g, cc):
                        s = idx[pl.ds(g * 16, 16)][lane]

                        @pl.when(s >= vocab)
                        def _():
                            pltpu.sync_copy(
                                new_v.at[pl.ds(s - vocab, 1)],
                                out_hbm.at[pl.ds(base + g * 16 + lane, 1)],
                            )

                        return cc

                    lax.fori_loop(0, CHUNK // 16, scan_groups, 0)

        def out_copy(t, buf):
            pltpu.sync_copy(buf, out_hbm.at[pl.ds(w_base + t * CHUNK, CHUNK)])

        acc_first = stage(0, idx0, safe0)
        fire(safe0, buf0, sem0)

        def pair_body(p, acc_cur):
            t0 = 2 * p
            acc_mid = stage(t0 + 1, idx1, safe1)
            fire(safe1, buf1, sem1)
            wait_gather(buf0, sem0)
            out_copy(t0, buf0)
            patch(t0, idx0, acc_cur)
            acc_nxt = stage(t0 + 2, idx0, safe0)
            fire(safe0, buf0, sem0)
            wait_gather(buf1, sem1)
            out_copy(t0 + 1, buf1)
            patch(t0 + 1, idx1, acc_mid)
            return acc_nxt

        acc_last = lax.fori_loop(0, n_pairs, pair_body, acc_first)
        wait_gather(buf0, sem0)
        out_copy(n_chunks - 1, buf0)
        patch(n_chunks - 1, idx0, acc_last)

    return k


def kernel(input_ids, orig_table, new_table):
    b, l = input_ids.shape
    vocab, dim = orig_table.shape
    n_new = new_table.shape[0]
    ids = input_ids.reshape(-1).astype(jnp.int32)
    k = _make_kernel(b * l, vocab, n_new, dim)
    out = k(ids, orig_table, new_table)
    return out.reshape(b, l, dim)
